# baseline (speedup 1.0000x reference)
import jax
import jax.numpy as jnp
from jax.experimental import pallas as pl
from jax.experimental.pallas import tpu as pltpu

_DIM = 384
_DIM_HEAD = 64
_HEADS = 8
_TOPK = 16
_H = 32
_W = 32
_P = _H * _W
_INNER = _DIM_HEAD * _HEADS


def _topk_select(score, jj, ii, tri):
    f32 = jnp.float32
    A = jnp.broadcast_to(score, (32, 32))
    At = jnp.transpose(A)
    beats = (A > At) | ((A == At) & (jj < ii))
    rank = jnp.sum(beats.astype(f32), axis=1, keepdims=True)
    maskf = (rank < float(_TOPK)).astype(f32)
    maskT = jnp.transpose(maskf)
    pos = jnp.sum(tri * maskT, axis=1, keepdims=True)
    posT = jnp.transpose(pos)
    slots = jax.lax.broadcasted_iota(jnp.int32, (_TOPK, 32), 0).astype(f32)
    S = (slots == jnp.broadcast_to(posT, (_TOPK, 32))).astype(f32)
    S = S * jnp.broadcast_to(maskT, (_TOPK, 32))
    return S


def _dpsa_body(x_ref, g_ref, b_ref, wqkv_ref, wout_ref, gamma_ref, y_ref):
    f32 = jnp.float32
    xb = x_ref[0]
    mean = jnp.mean(xb, axis=0, keepdims=True)
    xc = xb - mean
    var = jnp.mean(xc * xc, axis=0, keepdims=True)
    xn = xc * jax.lax.rsqrt(var + 1e-5) * g_ref[...] + b_ref[...]
    bf = jnp.bfloat16
    qkv = jnp.dot(wqkv_ref[...].astype(bf), xn.astype(bf),
                  preferred_element_type=f32)

    ph = jax.lax.broadcasted_iota(jnp.int32, (_P, _H), 0) // _W
    ch = jax.lax.broadcasted_iota(jnp.int32, (_P, _H), 1)
    er = (ph == ch).astype(f32)
    pw = jax.lax.broadcasted_iota(jnp.int32, (_P, _W), 0) % _W
    cw = jax.lax.broadcasted_iota(jnp.int32, (_P, _W), 1)
    ew = (pw == cw).astype(f32)
    nk = _TOPK * _TOPK
    ech = (jax.lax.broadcasted_iota(jnp.int32, (_TOPK, nk), 0)
           == jax.lax.broadcasted_iota(jnp.int32, (_TOPK, nk), 1) // _TOPK
           ).astype(f32)
    ecw = (jax.lax.broadcasted_iota(jnp.int32, (_TOPK, nk), 0)
           == jax.lax.broadcasted_iota(jnp.int32, (_TOPK, nk), 1) % _TOPK
           ).astype(f32)
    ii = jax.lax.broadcasted_iota(jnp.int32, (32, 32), 0)
    jj = jax.lax.broadcasted_iota(jnp.int32, (32, 32), 1)
    tri = (jj < ii).astype(f32)

    outs = []
    for h in range(_HEADS):
        qh = qkv[h * _DIM_HEAD:(h + 1) * _DIM_HEAD]
        kh = qkv[_INNER + h * _DIM_HEAD:_INNER + (h + 1) * _DIM_HEAD]
        vh = qkv[2 * _INNER + h * _DIM_HEAD:2 * _INNER + (h + 1) * _DIM_HEAD]
        qn = qh * (1.0 / jnp.maximum(
            jnp.sqrt(jnp.sum(qh * qh, axis=0, keepdims=True)), 1e-12))
        kn = kh * (1.0 / jnp.maximum(
            jnp.sqrt(jnp.sum(kh * kh, axis=0, keepdims=True)), 1e-12))
        k_abs = jnp.abs(kn)
        q_probe = jnp.sum(jnp.abs(qn), axis=1, keepdims=True)
        t = jnp.sum(q_probe * k_abs, axis=0, keepdims=True)
        score_r = jnp.dot(t, er, preferred_element_type=f32)
        score_c = jnp.dot(t, ew, preferred_element_type=f32)
        Sh = _topk_select(score_r, jj, ii, tri)
        Sw = _topk_select(score_c, jj, ii, tri)
        A0 = jnp.dot(er, jnp.transpose(Sh), preferred_element_type=f32)
        Aq = jnp.dot(A0, ech, preferred_element_type=f32)
        B0 = jnp.dot(ew, jnp.transpose(Sw), preferred_element_type=f32)
        Bq = jnp.dot(B0, ecw, preferred_element_type=f32)
        Qm = Aq * Bq
        kp = jnp.dot(kn.astype(bf), Qm.astype(bf),
                     preferred_element_type=f32)
        vp = jnp.dot(vh.astype(bf), Qm.astype(bf),
                     preferred_element_type=f32)
        sim = jax.lax.dot_general(qn.astype(bf), kp.astype(bf),
                                  (((0,), (0,)), ((), ())),
                                  preferred_element_type=f32)
        m = jnp.max(sim, axis=1, keepdims=True)
        e = jnp.exp(sim - m)
        attn = e / jnp.sum(e, axis=1, keepdims=True)
        oh = jax.lax.dot_general(attn.astype(bf), vp.astype(bf),
                                 (((1,), (1,)), ((), ())),
                                 preferred_element_type=f32)
        outs.append(oh)
    o = jnp.concatenate(outs, axis=1)
    out = jax.lax.dot_general(wout_ref[...].astype(bf), o.astype(bf),
                              (((1,), (1,)), ((), ())),
                              preferred_element_type=f32)
    y_ref[0] = gamma_ref[0, 0] * out + xn


def kernel(x, g, b, W_qkv, W_out, gamma):
    B, C, H, W = x.shape
    x2 = x.reshape(B, C, H * W)
    g2 = g.reshape(C, 1)
    b2 = b.reshape(C, 1)
    gm = jnp.asarray(gamma, jnp.float32).reshape(1, 1)
    y2 = pl.pallas_call(
        _dpsa_body,
        grid=(B,),
        in_specs=[
            pl.BlockSpec((1, C, H * W), lambda i: (i, 0, 0)),
            pl.BlockSpec((C, 1), lambda i: (0, 0)),
            pl.BlockSpec((C, 1), lambda i: (0, 0)),
            pl.BlockSpec(W_qkv.shape, lambda i: (0, 0)),
            pl.BlockSpec(W_out.shape, lambda i: (0, 0)),
            pl.BlockSpec((1, 1), lambda i: (0, 0)),
        ],
        out_specs=pl.BlockSpec((1, C, H * W), lambda i: (i, 0, 0)),
        out_shape=jax.ShapeDtypeStruct((B, C, H * W), jnp.float32),
        compiler_params=pltpu.CompilerParams(
            dimension_semantics=("parallel",)),
    )(x2, g2, b2, W_qkv, W_out, gm)
    return y2.reshape(B, C, H, W)


# speedup vs baseline: 1.3642x; 1.3642x over previous
import functools

import jax
import jax.numpy as jnp
from jax import lax
from jax.experimental import pallas as pl
from jax.experimental.pallas import tpu as pltpu
from jax.experimental.pallas import tpu_sc as plsc

_DIM = 384
_DIM_HEAD = 64
_HEADS = 8
_TOPK = 16
_H = 32
_W = 32
_P = _H * _W
_INNER = _DIM_HEAD * _HEADS
_NK = _TOPK * _TOPK


def _qkv_body(x_ref, g_ref, b_ref, wqkv_ref, qkvn_ref, xn_ref, sc_ref):
    f32 = jnp.float32
    xb = x_ref[0]
    mean = jnp.mean(xb, axis=0, keepdims=True)
    xc = xb - mean
    var = jnp.mean(xc * xc, axis=0, keepdims=True)
    xn = xc * lax.rsqrt(var + 1e-5) * g_ref[...] + b_ref[...]
    qkv = jnp.dot(wqkv_ref[...], xn, preferred_element_type=f32)

    ph = lax.broadcasted_iota(jnp.int32, (_P, _H), 0) // _W
    ch = lax.broadcasted_iota(jnp.int32, (_P, _H), 1)
    er = (ph == ch).astype(f32)
    pw = lax.broadcasted_iota(jnp.int32, (_P, _W), 0) % _W
    cw = lax.broadcasted_iota(jnp.int32, (_P, _W), 1)
    ew = (pw == cw).astype(f32)

    qs, ks, ss = [], [], []
    for h in range(_HEADS):
        qh = qkv[h * _DIM_HEAD:(h + 1) * _DIM_HEAD]
        kh = qkv[_INNER + h * _DIM_HEAD:_INNER + (h + 1) * _DIM_HEAD]
        qn = qh * (1.0 / jnp.maximum(
            jnp.sqrt(jnp.sum(qh * qh, axis=0, keepdims=True)), 1e-12))
        kn = kh * (1.0 / jnp.maximum(
            jnp.sqrt(jnp.sum(kh * kh, axis=0, keepdims=True)), 1e-12))
        q_probe = jnp.sum(jnp.abs(qn), axis=1, keepdims=True)
        t = jnp.sum(q_probe * jnp.abs(kn), axis=0, keepdims=True)
        score_r = jnp.dot(t, er, preferred_element_type=f32)
        score_c = jnp.dot(t, ew, preferred_element_type=f32)
        qs.append(qn)
        ks.append(kn)
        ss.append(jnp.concatenate([score_r, score_c], axis=1))
    qkvn_ref[0] = jnp.concatenate(qs + ks + [qkv[2 * _INNER:]], axis=0)
    xn_ref[0] = xn
    sc_ref[0] = jnp.concatenate(ss, axis=0)


def _beats(xj, ij, xi, ii):
    return jnp.where((xj > xi) | ((xj == xi) & (ij < ii)), 1.0, 0.0)


def _sc_topk_body(scores_hbm, out_hbm, sv, st):
    i32 = jnp.int32
    wid = lax.axis_index("s") * 2 + lax.axis_index("c")
    iota = lax.iota(i32, 16)

    def prefix_sum(v):
        cs = v
        for shift in (1, 2, 4, 8):
            idx = jnp.maximum(iota - shift, 0)
            g = cs.at[idx].get(mode="promise_in_bounds")
            cs = cs + jnp.where(iota >= shift, g, 0.0)
        return cs

    for u in range(2):
        bh = wid * 2 + u
        pltpu.sync_copy(scores_hbm.at[bh], sv)
        for half in range(2):
            s0 = sv[pl.ds(half * 32, 16)]
            s1 = sv[pl.ds(half * 32 + 16, 16)]
            i0 = iota
            i1 = iota + 16
            ra = jnp.zeros((16,), jnp.float32)
            rb = jnp.zeros((16,), jnp.float32)
            for k in range(16):
                perm = lax.rem(iota + k, 16)
                a_r = s0.at[perm].get(mode="promise_in_bounds")
                b_r = s1.at[perm].get(mode="promise_in_bounds")
                ia_r = perm
                ib_r = perm + 16
                if k > 0:
                    ra = ra + _beats(a_r, ia_r, s0, i0)
                    rb = rb + _beats(b_r, ib_r, s1, i1)
                ra = ra + _beats(b_r, ib_r, s0, i0)
                rb = rb + _beats(a_r, ia_r, s1, i1)
            mask_a = ra < float(_TOPK)
            mask_b = rb < float(_TOPK)
            ma = jnp.where(mask_a, 1.0, 0.0)
            mb = jnp.where(mask_b, 1.0, 0.0)
            cs_a = prefix_sum(ma)
            pos_a = cs_a - ma
            na = cs_a.at[iota * 0 + 15].get(mode="promise_in_bounds")
            cs_b = prefix_sum(mb)
            pos_b = cs_b - mb + na
            for s in range(16):
                sf = float(s)
                base = half * 512 + s * 32
                st[pl.ds(base, 16)] = jnp.where(
                    mask_a & (pos_a == sf), 1.0, 0.0)
                st[pl.ds(base + 16, 16)] = jnp.where(
                    mask_b & (pos_b == sf), 1.0, 0.0)
        pltpu.sync_copy(st, out_hbm.at[bh])


def _attn_body(qkvn_ref, st_ref, xn_ref, wout_ref, gamma_ref, y_ref):
    f32 = jnp.float32
    ech = (lax.broadcasted_iota(jnp.int32, (_TOPK, _NK), 0)
           == lax.broadcasted_iota(jnp.int32, (_TOPK, _NK), 1) // _TOPK
           ).astype(f32)
    ecw = (lax.broadcasted_iota(jnp.int32, (_TOPK, _NK), 0)
           == lax.broadcasted_iota(jnp.int32, (_TOPK, _NK), 1) % _TOPK
           ).astype(f32)
    outs = []
    for h in range(_HEADS):
        st_h = jnp.transpose(st_ref[0, h, 0])
        st_w = jnp.transpose(st_ref[0, h, 1])
        C1 = jnp.dot(st_h, ech, preferred_element_type=f32)
        C2 = jnp.dot(st_w, ecw, preferred_element_type=f32)
        Qm = (C1[:, None, :] * C2[None, :, :]).reshape(_P, _NK)
        qn = qkvn_ref[0, h * _DIM_HEAD:(h + 1) * _DIM_HEAD]
        kn = qkvn_ref[0, _INNER + h * _DIM_HEAD:
                      _INNER + (h + 1) * _DIM_HEAD]
        vh = qkvn_ref[0, 2 * _INNER + h * _DIM_HEAD:
                      2 * _INNER + (h + 1) * _DIM_HEAD]
        kv = jnp.concatenate([kn, vh], axis=0)
        kvp = jnp.dot(kv, Qm, preferred_element_type=f32)
        kp = kvp[:_DIM_HEAD]
        vp = kvp[_DIM_HEAD:]
        sim = lax.dot_general(qn, kp, (((0,), (0,)), ((), ())),
                              preferred_element_type=f32)
        e = jnp.exp(sim)
        esum = jnp.sum(e, axis=1, keepdims=True)
        oh = lax.dot_general(e, vp, (((1,), (1,)), ((), ())),
                             preferred_element_type=f32)
        outs.append(oh * (1.0 / esum))
    o = jnp.concatenate(outs, axis=1)
    out = lax.dot_general(wout_ref[...], o, (((1,), (1,)), ((), ())),
                          preferred_element_type=f32)
    y_ref[0] = gamma_ref[0, 0] * out + xn_ref[0]


def kernel(x, g, b, W_qkv, W_out, gamma):
    B, C, H, W = x.shape
    f32 = jnp.float32
    x2 = x.reshape(B, C, H * W)
    g2 = g.reshape(C, 1)
    b2 = b.reshape(C, 1)
    gm = jnp.asarray(gamma, f32).reshape(1, 1)

    qkvn, xn, scores = pl.pallas_call(
        _qkv_body,
        grid=(B,),
        in_specs=[
            pl.BlockSpec((1, C, H * W), lambda i: (i, 0, 0)),
            pl.BlockSpec((C, 1), lambda i: (0, 0)),
            pl.BlockSpec((C, 1), lambda i: (0, 0)),
            pl.BlockSpec(W_qkv.shape, lambda i: (0, 0)),
        ],
        out_specs=[
            pl.BlockSpec((1, 3 * _INNER, H * W), lambda i: (i, 0, 0)),
            pl.BlockSpec((1, C, H * W), lambda i: (i, 0, 0)),
            pl.BlockSpec((1, _HEADS, 64), lambda i: (i, 0, 0)),
        ],
        out_shape=[
            jax.ShapeDtypeStruct((B, 3 * _INNER, H * W), f32),
            jax.ShapeDtypeStruct((B, C, H * W), f32),
            jax.ShapeDtypeStruct((B, _HEADS, 64), f32),
        ],
        compiler_params=pltpu.CompilerParams(
            dimension_semantics=("parallel",)),
    )(x2, g2, b2, W_qkv)

    mesh = plsc.VectorSubcoreMesh(core_axis_name="c", subcore_axis_name="s")
    sc_topk = functools.partial(
        pl.kernel,
        mesh=mesh,
        out_type=jax.ShapeDtypeStruct((B * _HEADS, 1024), f32),
        scratch_types=[
            pltpu.VMEM((64,), f32),
            pltpu.VMEM((1024,), f32),
        ],
    )(_sc_topk_body)
    st = sc_topk(scores.reshape(B * _HEADS, 64))
    st4 = st.reshape(B, _HEADS, 2, 16, 32)

    y2 = pl.pallas_call(
        _attn_body,
        grid=(B,),
        in_specs=[
            pl.BlockSpec((1, 3 * _INNER, H * W), lambda i: (i, 0, 0)),
            pl.BlockSpec((1, _HEADS, 2, 16, 32), lambda i: (i, 0, 0, 0, 0)),
            pl.BlockSpec((1, C, H * W), lambda i: (i, 0, 0)),
            pl.BlockSpec(W_out.shape, lambda i: (0, 0)),
            pl.BlockSpec((1, 1), lambda i: (0, 0)),
        ],
        out_specs=pl.BlockSpec((1, C, H * W), lambda i: (i, 0, 0)),
        out_shape=jax.ShapeDtypeStruct((B, C, H * W), f32),
        compiler_params=pltpu.CompilerParams(
            dimension_semantics=("parallel",)),
    )(qkvn, st4, xn, W_out, gm)
    return y2.reshape(B, C, H, W)
